# baseline (speedup 1.0000x reference)
import functools

import jax
import jax.numpy as jnp
from jax import lax
from jax.experimental import pallas as pl
from jax.experimental.pallas import tpu as pltpu
from jax.experimental.pallas import tpu_sc as plsc

N = 10000
H = 256
B = 64
T = 1000

NP = 10240
BLK = 512
NBLK = NP // BLK
PN = N

NS = 16
HH = 128
RB = 128
NB = 84
CE = NB * RB
EP = NS * CE
RED = NP // NS
E_TOT = 160000 + N


def _temb_body(t_ref, emb_ref, out_ref):
    tt = t_ref[...]
    oh = (lax.broadcasted_iota(jnp.int32, (B, T), 1) == tt).astype(jnp.float32)
    out_ref[...] = jnp.dot(oh, emb_ref[...], preferred_element_type=jnp.float32)


def _scores(h, asrc, adst):
    a_s = lax.dot_general(asrc, h, (((1,), (1,)), ((), ())),
                          preferred_element_type=jnp.float32)
    a_d = lax.dot_general(adst, h, (((1,), (1,)), ((), ())),
                          preferred_element_type=jnp.float32)
    z = jnp.zeros((7, BLK), jnp.float32)
    return jnp.concatenate([a_s, z, a_d, z], axis=0)


def _dense1_body(x_ref, bat_ref, temb_ref, wi_ref, bi_ref, w_ref, asrc_ref,
                 adst_ref, hl_ref, hr_ref, asad_ref):
    bidx = bat_ref[...]
    oh = (lax.broadcasted_iota(jnp.int32, (BLK, B), 1) == bidx).astype(jnp.float32)
    x = (jnp.dot(x_ref[...], wi_ref[...], preferred_element_type=jnp.float32)
         + bi_ref[...]
         + jnp.dot(oh, temb_ref[...], preferred_element_type=jnp.float32))
    h = jnp.dot(x, w_ref[...], preferred_element_type=jnp.float32)
    hl_ref[...] = h[:, :HH]
    hr_ref[...] = h[:, HH:]
    asad_ref[...] = _scores(h, asrc_ref[...], adst_ref[...])


def _dense2_body(al_ref, ar_ref, b1_ref, w_ref, asrc_ref, adst_ref,
                 hl_ref, hr_ref, asad_ref):
    x = jnp.concatenate([al_ref[...], ar_ref[...]], axis=1) + b1_ref[...]
    x = jnp.maximum(x, 0.0)
    h = jnp.dot(x, w_ref[...], preferred_element_type=jnp.float32)
    hl_ref[...] = h[:, :HH]
    hr_ref[...] = h[:, HH:]
    asad_ref[...] = _scores(h, asrc_ref[...], adst_ref[...])


def _dense3_body(al_ref, ar_ref, b2_ref, wo_ref, bo_ref, y_ref):
    x = jnp.concatenate([al_ref[...], ar_ref[...]], axis=1) + b2_ref[...]
    y_ref[...] = jnp.dot(x, wo_ref[...], preferred_element_type=jnp.float32) + bo_ref[...]


def _zero_rows(ref, nrows):
    def row(i, _):
        for c in range(ref.shape[1] // 16):
            ref[i, pl.ds(c * 16, 16)] = jnp.zeros((16,), jnp.float32)
        return 0
    lax.fori_loop(0, nrows, row, 0)


ZB = 64


def _zero_acc_slice(sid, zbuf, acc_sh, rpt):
    def row(i, _):
        for c in range(HH // 16):
            zbuf[0, i, pl.ds(c * 16, 16)] = jnp.zeros((16,), jnp.float32)
        return 0
    lax.fori_loop(0, ZB, row, 0)
    for k in range(rpt // ZB):
        pltpu.sync_copy(zbuf.at[0, pl.ds(0, ZB)],
                        acc_sh.at[pl.ds(sid * rpt + k * ZB, ZB)])


KB = 2


def _phase_b(sid, src_v, dst_v, ee_v, den_v, gbuf, gsem,
             alpha_v, dstl_v, acc_sh, h_hbm, agg_hbm, lo, nrows):
    def batch(p, _):
        b0 = p * KB

        def fire(k, _):
            pltpu.async_copy(h_hbm.at[src_v.at[b0 + k]], gbuf.at[k], gsem)
            return 0
        lax.fori_loop(0, KB, fire, 0)

        def drainwait(k, _):
            pltpu.make_async_copy(
                h_hbm.at[src_v.at[b0 + k]], gbuf.at[k], gsem).wait()
            return 0
        lax.fori_loop(0, KB, drainwait, 0)

        def proc(k, _):
            b = b0 + k

            def asub(j, _):
                d16 = dst_v[b, pl.ds(j * 16, 16)]
                den = plsc.load_gather(den_v, [d16])
                ee = ee_v[b, pl.ds(j * 16, 16)]
                dloc = d16 - lo
                inr = (dloc >= 0) & (dloc < nrows)
                alpha_v[pl.ds(j * 16, 16)] = jnp.where(
                    inr, ee / (den + 1e-16), 0.0)
                dstl_v[k, pl.ds(j * 16, 16)] = jnp.where(inr, dloc, 0)
                return 0
            lax.fori_loop(0, RB // 16, asub, 0)

            def rsub(g, _):
                a16 = alpha_v[pl.ds(g * 16, 16)]
                for l in range(16):
                    av = jnp.full((16,), a16[l], jnp.float32)
                    r = g * 16 + l
                    for c in range(HH // 16):
                        gbuf[k, r, pl.ds(c * 16, 16)] = (
                            gbuf[k, r, pl.ds(c * 16, 16)] * av)
                return 0
            lax.fori_loop(0, RB // 16, rsub, 0)

            pltpu.sync_copy(gbuf.at[k], acc_sh.at[dstl_v.at[k]], add=True)
            return 0
        lax.fori_loop(0, KB, proc, 0)
        return 0
    lax.fori_loop(0, NB // KB, batch, 0)

    plsc.subcore_barrier()
    rpt = nrows // NS
    pltpu.sync_copy(acc_sh.at[pl.ds(sid * rpt, rpt)],
                    agg_hbm.at[pl.ds(sid * rpt, rpt)])


def _gat_soft_body(src_hbm, dst_hbm, asad_hbm, ee_hbm, den_hbm,
                   src_v, dst_v, ee_v, as_v, ad_v, den_v, red_v, red2_v,
                   den_sh, den2_sh):
    sid = lax.axis_index("s")

    pltpu.sync_copy(src_hbm.at[sid], src_v)
    pltpu.sync_copy(dst_hbm.at[sid], dst_v)
    pltpu.sync_copy(asad_hbm.at[0], as_v)
    pltpu.sync_copy(asad_hbm.at[8], ad_v)

    def zden(i, _):
        den_v[pl.ds(i * 16, 16)] = jnp.zeros((16,), jnp.float32)
        return 0
    lax.fori_loop(0, NP // 16, zden, 0)

    def chunk_a(b, _):
        def sub(j, _):
            s16 = src_v[b, pl.ds(j * 16, 16)]
            d16 = dst_v[b, pl.ds(j * 16, 16)]
            sc = plsc.load_gather(as_v, [s16]) + plsc.load_gather(ad_v, [d16])
            sc = jnp.where(sc >= 0, sc, 0.2 * sc)
            ee = jnp.exp(sc)
            ee_v[b, pl.ds(j * 16, 16)] = ee
            plsc.addupdate_scatter(den_v, [d16], ee)
            return 0
        lax.fori_loop(0, RB // 16, sub, 0)
        return 0
    lax.fori_loop(0, NB, chunk_a, 0)

    pltpu.sync_copy(den_v, den_sh.at[sid])
    plsc.subcore_barrier()
    pltpu.sync_copy(den_sh.at[:, pl.ds(sid * RED, RED)], red_v)

    def red_body(j, _):
        acc = red_v[0, pl.ds(j * 16, 16)]
        for t in range(1, NS):
            acc = acc + red_v[t, pl.ds(j * 16, 16)]
        red2_v[pl.ds(j * 16, 16)] = acc
        return 0
    lax.fori_loop(0, RED // 16, red_body, 0)
    pltpu.sync_copy(red2_v, den2_sh.at[pl.ds(sid * RED, RED)])

    pltpu.sync_copy(ee_v, ee_hbm.at[sid])
    pltpu.sync_copy(red2_v, den_hbm.at[pl.ds(sid * RED, RED)])


def _gat_agg_body(src_hbm, dst_hbm, ee_hbm, den_hbm, h_hbm, agg_hbm,
                  src_v, dst_v, ee_v, den_v, gbuf,
                  alpha_v, dstl_v, acc_sh, gsem,
                  *, lo, nrows):
    sid = lax.axis_index("s")

    pltpu.sync_copy(src_hbm.at[sid], src_v)
    pltpu.sync_copy(dst_hbm.at[sid], dst_v)
    pltpu.sync_copy(ee_hbm.at[sid], ee_v)
    pltpu.sync_copy(den_hbm, den_v)

    _zero_acc_slice(sid, gbuf, acc_sh, nrows // NS)
    plsc.subcore_barrier()

    _phase_b(sid, src_v, dst_v, ee_v, den_v, gbuf, gsem,
             alpha_v, dstl_v, acc_sh, h_hbm, agg_hbm, lo, nrows)


@functools.lru_cache(maxsize=1)
def _build_sc_kernels():
    mesh = plsc.VectorSubcoreMesh(core_axis_name="c", subcore_axis_name="s",
                                  num_cores=1)
    f32 = jnp.float32
    agg_t = jax.ShapeDtypeStruct((NP, HH), f32)
    common_v = [
        pltpu.VMEM((NB, RB), jnp.int32),
        pltpu.VMEM((NB, RB), jnp.int32),
        pltpu.VMEM((NB, RB), f32),
    ]
    NSPLIT = 5120
    soft = functools.partial(
        pl.kernel,
        mesh=mesh,
        out_type=[jax.ShapeDtypeStruct((NS, NB, RB), f32),
                  jax.ShapeDtypeStruct((NP,), f32)],
        scratch_types=common_v + [
            pltpu.VMEM((NP,), f32),
            pltpu.VMEM((NP,), f32),
            pltpu.VMEM((NP,), f32),
            pltpu.VMEM((NS, RED), f32),
            pltpu.VMEM((RED,), f32),
            pltpu.VMEM_SHARED((NS, NP), f32),
            pltpu.VMEM_SHARED((NP,), f32),
        ],
        compiler_params=pltpu.CompilerParams(needs_layout_passes=False),
    )(_gat_soft_body)

    def make_agg(lo, nrows):
        body = functools.partial(_gat_agg_body, lo=lo, nrows=nrows)
        return functools.partial(
            pl.kernel,
            mesh=mesh,
            out_type=[jax.ShapeDtypeStruct((nrows, HH), f32)],
            scratch_types=common_v + [
                pltpu.VMEM((NP,), f32),
                pltpu.VMEM((KB, RB, HH), f32),
                pltpu.VMEM((RB,), f32),
                pltpu.VMEM((8, RB), jnp.int32),
                pltpu.VMEM_SHARED((nrows, HH), f32),
                pltpu.SemaphoreType.DMA,
            ],
            compiler_params=pltpu.CompilerParams(needs_layout_passes=False),
        )(body)

    return soft, make_agg(0, NSPLIT), make_agg(NSPLIT, NP - NSPLIT)


def _gat_edge(src2, dst2, asad, hl, hr):
    soft, agg_a, agg_b = _build_sc_kernels()
    ee, den = soft(src2, dst2, asad)
    (aggl_a,) = agg_a(src2, dst2, ee, den, hl)
    (aggr_a,) = agg_a(src2, dst2, ee, den, hr)
    (aggl_b,) = agg_b(src2, dst2, ee, den, hl)
    (aggr_b,) = agg_b(src2, dst2, ee, den, hr)
    aggl = jnp.concatenate([aggl_a, aggl_b], axis=0)
    aggr = jnp.concatenate([aggr_a, aggr_b], axis=0)
    return aggl, aggr


def kernel(X_t, t, edge_index, batch, emb_t, W_in, b_in, W1, a_src1, a_dst1,
           b1, W2, a_src2, a_dst2, b2, W_out, b_out):
    f32 = jnp.float32

    xp = jnp.pad(X_t, ((0, NP - N), (0, 5)))
    wi = jnp.pad(W_in, ((0, 5), (0, 0)))
    bat2 = jnp.pad(batch, (0, NP - N)).reshape(NP, 1).astype(jnp.int32)
    t2 = t.reshape(B, 1).astype(jnp.int32)

    loop = jnp.arange(N, dtype=jnp.int32)
    epad = jnp.full((EP - E_TOT,), PN, jnp.int32)
    src2 = jnp.concatenate([edge_index[0].astype(jnp.int32), loop, epad]
                           ).reshape(NS, NB, RB)
    dst2 = jnp.concatenate([edge_index[1].astype(jnp.int32), loop, epad]
                           ).reshape(NS, NB, RB)

    wo = jnp.pad(W_out, ((0, 0), (0, 125)))
    bo = jnp.pad(b_out, (0, 125))[None, :]

    temb = pl.pallas_call(
        _temb_body,
        out_shape=jax.ShapeDtypeStruct((B, H), f32),
    )(t2, emb_t)

    full = lambda s: pl.BlockSpec(s, lambda i: (0, 0))
    rowb = lambda s: pl.BlockSpec(s, lambda i: (i, 0))
    asadb = pl.BlockSpec((16, BLK), lambda i: (0, i))
    hout = [jax.ShapeDtypeStruct((NP, HH), f32)] * 2 + [
        jax.ShapeDtypeStruct((16, NP), f32)]
    hspecs = [rowb((BLK, HH))] * 2 + [asadb]

    hl1, hr1, asad1 = pl.pallas_call(
        _dense1_body,
        grid=(NBLK,),
        in_specs=[rowb((BLK, 8)), rowb((BLK, 1)), full((B, H)), full((8, H)),
                  full((1, H)), full((H, H)), full((1, H)), full((1, H))],
        out_specs=hspecs,
        out_shape=hout,
    )(xp, bat2, temb, wi, b_in[None, :], W1, a_src1[None, :], a_dst1[None, :])

    aggl1, aggr1 = _gat_edge(src2, dst2, asad1, hl1, hr1)

    hl2, hr2, asad2 = pl.pallas_call(
        _dense2_body,
        grid=(NBLK,),
        in_specs=[rowb((BLK, HH))] * 2 + [full((1, H)), full((H, H)),
                                          full((1, H)), full((1, H))],
        out_specs=hspecs,
        out_shape=hout,
    )(aggl1, aggr1, b1[None, :], W2, a_src2[None, :], a_dst2[None, :])

    aggl2, aggr2 = _gat_edge(src2, dst2, asad2, hl2, hr2)

    y = pl.pallas_call(
        _dense3_body,
        grid=(NBLK,),
        in_specs=[rowb((BLK, HH))] * 2 + [full((1, H)), full((H, 128)),
                                          full((1, 128))],
        out_specs=rowb((BLK, 128)),
        out_shape=jax.ShapeDtypeStruct((NP, 128), f32),
    )(aggl2, aggr2, b2[None, :], wo, bo)

    return y[:N, :3]


# speedup vs baseline: 1.8323x; 1.8323x over previous
import functools

import jax
import jax.numpy as jnp
from jax import lax
from jax.experimental import pallas as pl
from jax.experimental.pallas import tpu as pltpu
from jax.experimental.pallas import tpu_sc as plsc

N = 10000
H = 256
B = 64
T = 1000

NP = 10240
BLK = 512
NBLK = NP // BLK
PN = N

NS = 16
HH = 128
RB = 128
NB = 84
CE = NB * RB
EP = NS * CE
RED = NP // NS
E_TOT = 160000 + N


def _temb_body(t_ref, emb_ref, out_ref):
    tt = t_ref[...]
    oh = (lax.broadcasted_iota(jnp.int32, (B, T), 1) == tt).astype(jnp.float32)
    out_ref[...] = jnp.dot(oh, emb_ref[...], preferred_element_type=jnp.float32)


def _scores(h, asrc, adst):
    a_s = lax.dot_general(asrc, h, (((1,), (1,)), ((), ())),
                          preferred_element_type=jnp.float32)
    a_d = lax.dot_general(adst, h, (((1,), (1,)), ((), ())),
                          preferred_element_type=jnp.float32)
    z = jnp.zeros((7, BLK), jnp.float32)
    return jnp.concatenate([a_s, z, a_d, z], axis=0)


def _dense1_body(x_ref, bat_ref, temb_ref, wi_ref, bi_ref, w_ref, asrc_ref,
                 adst_ref, hl_ref, hr_ref, asad_ref):
    bidx = bat_ref[...]
    oh = (lax.broadcasted_iota(jnp.int32, (BLK, B), 1) == bidx).astype(jnp.float32)
    x = (jnp.dot(x_ref[...], wi_ref[...], preferred_element_type=jnp.float32)
         + bi_ref[...]
         + jnp.dot(oh, temb_ref[...], preferred_element_type=jnp.float32))
    h = jnp.dot(x, w_ref[...], preferred_element_type=jnp.float32)
    hl_ref[...] = h[:, :HH]
    hr_ref[...] = h[:, HH:]
    asad_ref[...] = _scores(h, asrc_ref[...], adst_ref[...])


def _dense2_body(al_ref, ar_ref, b1_ref, w_ref, asrc_ref, adst_ref,
                 hl_ref, hr_ref, asad_ref):
    x = jnp.concatenate([al_ref[...], ar_ref[...]], axis=1) + b1_ref[...]
    x = jnp.maximum(x, 0.0)
    h = jnp.dot(x, w_ref[...], preferred_element_type=jnp.float32)
    hl_ref[...] = h[:, :HH]
    hr_ref[...] = h[:, HH:]
    asad_ref[...] = _scores(h, asrc_ref[...], adst_ref[...])


def _dense3_body(al_ref, ar_ref, b2_ref, wo_ref, bo_ref, y_ref):
    x = jnp.concatenate([al_ref[...], ar_ref[...]], axis=1) + b2_ref[...]
    y_ref[...] = jnp.dot(x, wo_ref[...], preferred_element_type=jnp.float32) + bo_ref[...]


def _zero_rows(ref, nrows):
    def row(i, _):
        for c in range(ref.shape[1] // 16):
            ref[i, pl.ds(c * 16, 16)] = jnp.zeros((16,), jnp.float32)
        return 0
    lax.fori_loop(0, nrows, row, 0)


ZB = 64


def _zero_acc_slice(sid, zbuf, acc_sh, rpt):
    def row(i, _):
        for c in range(HH // 16):
            zbuf[i, pl.ds(c * 16, 16)] = jnp.zeros((16,), jnp.float32)
        return 0
    lax.fori_loop(0, ZB, row, 0)
    for k in range(rpt // ZB):
        pltpu.sync_copy(zbuf.at[pl.ds(0, ZB)],
                        acc_sh.at[pl.ds(sid * rpt + k * ZB, ZB)])


def _phase_b(sid, src_v, dst_v, ee_v, den_v, rows_v, gsem,
             alpha_v, dstl_v, acc_sh, h_hbm, agg_hbm, lo, nrows):
    def chunk_b(b, _):
        pltpu.async_copy(h_hbm.at[src_v.at[b]], rows_v, gsem).wait()

        def asub(j, _):
            d16 = dst_v[b, pl.ds(j * 16, 16)]
            den = plsc.load_gather(den_v, [d16])
            ee = ee_v[b, pl.ds(j * 16, 16)]
            dloc = d16 - lo
            inr = (dloc >= 0) & (dloc < nrows)
            alpha_v[pl.ds(j * 16, 16)] = jnp.where(
                inr, ee / (den + 1e-16), 0.0)
            dstl_v[0, pl.ds(j * 16, 16)] = jnp.where(inr, dloc, 0)
            return 0
        lax.fori_loop(0, RB // 16, asub, 0)

        def rsub(g, _):
            a16 = alpha_v[pl.ds(g * 16, 16)]
            for l in range(16):
                av = jnp.full((16,), a16[l], jnp.float32)
                r = g * 16 + l
                for c in range(HH // 16):
                    rows_v[r, pl.ds(c * 16, 16)] = (
                        rows_v[r, pl.ds(c * 16, 16)] * av)
            return 0
        lax.fori_loop(0, RB // 16, rsub, 0)

        pltpu.sync_copy(rows_v, acc_sh.at[dstl_v.at[0]], add=True)
        return 0
    lax.fori_loop(0, NB, chunk_b, 0)

    plsc.subcore_barrier()
    rpt = nrows // NS
    pltpu.sync_copy(acc_sh.at[pl.ds(sid * rpt, rpt)],
                    agg_hbm.at[pl.ds(sid * rpt, rpt)])


def _gat_soft_body(src_hbm, dst_hbm, asad_hbm, ee_hbm, den_hbm,
                   src_v, dst_v, ee_v, as_v, ad_v, den_v, red_v, red2_v,
                   den_sh, den2_sh):
    sid = lax.axis_index("s")

    pltpu.sync_copy(src_hbm.at[sid], src_v)
    pltpu.sync_copy(dst_hbm.at[sid], dst_v)
    pltpu.sync_copy(asad_hbm.at[0], as_v)
    pltpu.sync_copy(asad_hbm.at[8], ad_v)

    def zden(i, _):
        den_v[pl.ds(i * 16, 16)] = jnp.zeros((16,), jnp.float32)
        return 0
    lax.fori_loop(0, NP // 16, zden, 0)

    def chunk_a(b, _):
        def sub(j, _):
            s16 = src_v[b, pl.ds(j * 16, 16)]
            d16 = dst_v[b, pl.ds(j * 16, 16)]
            sc = plsc.load_gather(as_v, [s16]) + plsc.load_gather(ad_v, [d16])
            sc = jnp.where(sc >= 0, sc, 0.2 * sc)
            ee = jnp.exp(sc)
            ee_v[b, pl.ds(j * 16, 16)] = ee
            plsc.addupdate_scatter(den_v, [d16], ee)
            return 0
        lax.fori_loop(0, RB // 16, sub, 0)
        return 0
    lax.fori_loop(0, NB, chunk_a, 0)

    pltpu.sync_copy(den_v, den_sh.at[sid])
    plsc.subcore_barrier()
    pltpu.sync_copy(den_sh.at[:, pl.ds(sid * RED, RED)], red_v)

    def red_body(j, _):
        acc = red_v[0, pl.ds(j * 16, 16)]
        for t in range(1, NS):
            acc = acc + red_v[t, pl.ds(j * 16, 16)]
        red2_v[pl.ds(j * 16, 16)] = acc
        return 0
    lax.fori_loop(0, RED // 16, red_body, 0)
    pltpu.sync_copy(red2_v, den2_sh.at[pl.ds(sid * RED, RED)])

    pltpu.sync_copy(ee_v, ee_hbm.at[sid])
    pltpu.sync_copy(red2_v, den_hbm.at[pl.ds(sid * RED, RED)])


def _gat_agg_body(src_hbm, dst_hbm, ee_hbm, den_hbm, h_hbm, agg_hbm,
                  src_v, dst_v, ee_v, den_v, rows_v,
                  alpha_v, dstl_v, acc_sh, gsem,
                  *, lo, nrows):
    sid = lax.axis_index("s")

    pltpu.sync_copy(src_hbm.at[sid], src_v)
    pltpu.sync_copy(dst_hbm.at[sid], dst_v)
    pltpu.sync_copy(ee_hbm.at[sid], ee_v)
    pltpu.sync_copy(den_hbm, den_v)

    _zero_acc_slice(sid, rows_v, acc_sh, nrows // NS)
    plsc.subcore_barrier()

    _phase_b(sid, src_v, dst_v, ee_v, den_v, rows_v, gsem,
             alpha_v, dstl_v, acc_sh, h_hbm, agg_hbm, lo, nrows)


@functools.lru_cache(maxsize=1)
def _build_sc_kernels():
    mesh = plsc.VectorSubcoreMesh(core_axis_name="c", subcore_axis_name="s",
                                  num_cores=1)
    f32 = jnp.float32
    agg_t = jax.ShapeDtypeStruct((NP, HH), f32)
    common_v = [
        pltpu.VMEM((NB, RB), jnp.int32),
        pltpu.VMEM((NB, RB), jnp.int32),
        pltpu.VMEM((NB, RB), f32),
    ]
    NSPLIT = 8192
    soft = functools.partial(
        pl.kernel,
        mesh=mesh,
        out_type=[jax.ShapeDtypeStruct((NS, NB, RB), f32),
                  jax.ShapeDtypeStruct((NP,), f32)],
        scratch_types=common_v + [
            pltpu.VMEM((NP,), f32),
            pltpu.VMEM((NP,), f32),
            pltpu.VMEM((NP,), f32),
            pltpu.VMEM((NS, RED), f32),
            pltpu.VMEM((RED,), f32),
            pltpu.VMEM_SHARED((NS, NP), f32),
            pltpu.VMEM_SHARED((NP,), f32),
        ],
        compiler_params=pltpu.CompilerParams(needs_layout_passes=False),
    )(_gat_soft_body)

    def make_agg(lo, nrows):
        body = functools.partial(_gat_agg_body, lo=lo, nrows=nrows)
        return functools.partial(
            pl.kernel,
            mesh=mesh,
            out_type=[jax.ShapeDtypeStruct((nrows, HH), f32)],
            scratch_types=common_v + [
                pltpu.VMEM((NP,), f32),
                pltpu.VMEM((RB, HH), f32),
                pltpu.VMEM((RB,), f32),
                pltpu.VMEM((8, RB), jnp.int32),
                pltpu.VMEM_SHARED((nrows, HH), f32),
                pltpu.SemaphoreType.DMA,
            ],
            compiler_params=pltpu.CompilerParams(needs_layout_passes=False),
        )(body)

    return soft, make_agg(0, NSPLIT), make_agg(NSPLIT, NP - NSPLIT)


def _gat_edge(src2, dst2, asad, hl, hr):
    soft, agg_a, agg_b = _build_sc_kernels()
    ee, den = soft(src2, dst2, asad)
    (aggl_a,) = agg_a(src2, dst2, ee, den, hl)
    (aggr_a,) = agg_a(src2, dst2, ee, den, hr)
    (aggl_b,) = agg_b(src2, dst2, ee, den, hl)
    (aggr_b,) = agg_b(src2, dst2, ee, den, hr)
    aggl = jnp.concatenate([aggl_a, aggl_b], axis=0)
    aggr = jnp.concatenate([aggr_a, aggr_b], axis=0)
    return aggl, aggr


def kernel(X_t, t, edge_index, batch, emb_t, W_in, b_in, W1, a_src1, a_dst1,
           b1, W2, a_src2, a_dst2, b2, W_out, b_out):
    f32 = jnp.float32

    xp = jnp.pad(X_t, ((0, NP - N), (0, 5)))
    wi = jnp.pad(W_in, ((0, 5), (0, 0)))
    bat2 = jnp.pad(batch, (0, NP - N)).reshape(NP, 1).astype(jnp.int32)
    t2 = t.reshape(B, 1).astype(jnp.int32)

    loop = jnp.arange(N, dtype=jnp.int32)
    epad = jnp.full((EP - E_TOT,), PN, jnp.int32)
    src2 = jnp.concatenate([edge_index[0].astype(jnp.int32), loop, epad]
                           ).reshape(NS, NB, RB)
    dst2 = jnp.concatenate([edge_index[1].astype(jnp.int32), loop, epad]
                           ).reshape(NS, NB, RB)

    wo = jnp.pad(W_out, ((0, 0), (0, 125)))
    bo = jnp.pad(b_out, (0, 125))[None, :]

    temb = pl.pallas_call(
        _temb_body,
        out_shape=jax.ShapeDtypeStruct((B, H), f32),
    )(t2, emb_t)

    full = lambda s: pl.BlockSpec(s, lambda i: (0, 0))
    rowb = lambda s: pl.BlockSpec(s, lambda i: (i, 0))
    asadb = pl.BlockSpec((16, BLK), lambda i: (0, i))
    hout = [jax.ShapeDtypeStruct((NP, HH), f32)] * 2 + [
        jax.ShapeDtypeStruct((16, NP), f32)]
    hspecs = [rowb((BLK, HH))] * 2 + [asadb]

    hl1, hr1, asad1 = pl.pallas_call(
        _dense1_body,
        grid=(NBLK,),
        in_specs=[rowb((BLK, 8)), rowb((BLK, 1)), full((B, H)), full((8, H)),
                  full((1, H)), full((H, H)), full((1, H)), full((1, H))],
        out_specs=hspecs,
        out_shape=hout,
    )(xp, bat2, temb, wi, b_in[None, :], W1, a_src1[None, :], a_dst1[None, :])

    aggl1, aggr1 = _gat_edge(src2, dst2, asad1, hl1, hr1)

    hl2, hr2, asad2 = pl.pallas_call(
        _dense2_body,
        grid=(NBLK,),
        in_specs=[rowb((BLK, HH))] * 2 + [full((1, H)), full((H, H)),
                                          full((1, H)), full((1, H))],
        out_specs=hspecs,
        out_shape=hout,
    )(aggl1, aggr1, b1[None, :], W2, a_src2[None, :], a_dst2[None, :])

    aggl2, aggr2 = _gat_edge(src2, dst2, asad2, hl2, hr2)

    y = pl.pallas_call(
        _dense3_body,
        grid=(NBLK,),
        in_specs=[rowb((BLK, HH))] * 2 + [full((1, H)), full((H, 128)),
                                          full((1, 128))],
        out_specs=rowb((BLK, 128)),
        out_shape=jax.ShapeDtypeStruct((NP, 128), f32),
    )(aggl2, aggr2, b2[None, :], wo, bo)

    return y[:N, :3]


# speedup vs baseline: 2.3943x; 1.3067x over previous
import functools

import jax
import jax.numpy as jnp
from jax import lax
from jax.experimental import pallas as pl
from jax.experimental.pallas import tpu as pltpu
from jax.experimental.pallas import tpu_sc as plsc

N = 10000
H = 256
B = 64
T = 1000

NP = 10240
BLK = 512
NBLK = NP // BLK
PN = N

NS = 16
HH = 128
RB = 128
NB = 84
CE = NB * RB
EP = NS * CE
RED = NP // NS
E_TOT = 160000 + N


def _temb_body(t_ref, emb_ref, out_ref):
    tt = t_ref[...]
    oh = (lax.broadcasted_iota(jnp.int32, (B, T), 1) == tt).astype(jnp.float32)
    out_ref[...] = jnp.dot(oh, emb_ref[...], preferred_element_type=jnp.float32)


def _scores(h, asrc, adst):
    a_s = lax.dot_general(asrc, h, (((1,), (1,)), ((), ())),
                          preferred_element_type=jnp.float32)
    a_d = lax.dot_general(adst, h, (((1,), (1,)), ((), ())),
                          preferred_element_type=jnp.float32)
    z = jnp.zeros((7, BLK), jnp.float32)
    return jnp.concatenate([a_s, z, a_d, z], axis=0)


def _dense1_body(x_ref, bat_ref, temb_ref, wi_ref, bi_ref, w_ref, asrc_ref,
                 adst_ref, hl_ref, hr_ref, asad_ref):
    bidx = bat_ref[...]
    oh = (lax.broadcasted_iota(jnp.int32, (BLK, B), 1) == bidx).astype(jnp.float32)
    x = (jnp.dot(x_ref[...], wi_ref[...], preferred_element_type=jnp.float32)
         + bi_ref[...]
         + jnp.dot(oh, temb_ref[...], preferred_element_type=jnp.float32))
    h = jnp.dot(x, w_ref[...], preferred_element_type=jnp.float32)
    hl_ref[...] = h[:, :HH]
    hr_ref[...] = h[:, HH:]
    asad_ref[...] = _scores(h, asrc_ref[...], adst_ref[...])


def _dense2_body(al_ref, ar_ref, b1_ref, w_ref, asrc_ref, adst_ref,
                 hl_ref, hr_ref, asad_ref):
    x = jnp.concatenate([al_ref[...], ar_ref[...]], axis=1) + b1_ref[...]
    x = jnp.maximum(x, 0.0)
    h = jnp.dot(x, w_ref[...], preferred_element_type=jnp.float32)
    hl_ref[...] = h[:, :HH]
    hr_ref[...] = h[:, HH:]
    asad_ref[...] = _scores(h, asrc_ref[...], adst_ref[...])


def _dense3_body(al_ref, ar_ref, b2_ref, wo_ref, bo_ref, y_ref):
    x = jnp.concatenate([al_ref[...], ar_ref[...]], axis=1) + b2_ref[...]
    y_ref[...] = jnp.dot(x, wo_ref[...], preferred_element_type=jnp.float32) + bo_ref[...]


ZB = 64


def _zero_acc_slice(sid, zbuf, acc_sh, rpt):
    def row(i, _):
        for c in range(HH // 16):
            zbuf[i, pl.ds(c * 16, 16)] = jnp.zeros((16,), jnp.float32)
        return 0
    lax.fori_loop(0, ZB, row, 0)
    for k in range(rpt // ZB):
        pltpu.sync_copy(zbuf.at[pl.ds(0, ZB)],
                        acc_sh.at[pl.ds(sid * rpt + k * ZB, ZB)])


def _phase_b(sid, src_v, dst_v, ee_v, den_v, rows0, rows1, gs0, gs1,
             alpha_v, dstl_v, acc_sh, h_hbm, agg_hbm, lo, nrows):
    pltpu.async_copy(h_hbm.at[src_v.at[0]], rows0, gs0)
    pltpu.async_copy(h_hbm.at[src_v.at[1]], rows1, gs1)

    def pair(p, _):
        for k, rows_v, gsem in ((0, rows0, gs0), (1, rows1, gs1)):
            b = 2 * p + k
            pltpu.make_async_copy(h_hbm.at[src_v.at[b]], rows_v, gsem).wait()

            def asub(j, _):
                d16 = dst_v[b, pl.ds(j * 16, 16)]
                den = plsc.load_gather(den_v, [d16])
                ee = ee_v[b, pl.ds(j * 16, 16)]
                dloc = d16 - lo
                inr = (dloc >= 0) & (dloc < nrows)
                alpha_v[pl.ds(j * 16, 16)] = jnp.where(
                    inr, ee / (den + 1e-16), 0.0)
                dstl_v[0, pl.ds(j * 16, 16)] = jnp.where(inr, dloc, 0)
                return 0
            lax.fori_loop(0, RB // 16, asub, 0)

            def rsub(g, _):
                a16 = alpha_v[pl.ds(g * 16, 16)]
                for l in range(16):
                    av = jnp.full((16,), a16[l], jnp.float32)
                    r = g * 16 + l
                    for c in range(HH // 16):
                        rows_v[r, pl.ds(c * 16, 16)] = (
                            rows_v[r, pl.ds(c * 16, 16)] * av)
                return 0
            lax.fori_loop(0, RB // 16, rsub, 0)

            pltpu.sync_copy(rows_v, acc_sh.at[dstl_v.at[0]], add=True)

            @pl.when(b + 2 < NB)
            def _():
                pltpu.async_copy(h_hbm.at[src_v.at[b + 2]], rows_v, gsem)
        return 0
    lax.fori_loop(0, NB // 2, pair, 0)

    plsc.subcore_barrier()
    rpt = nrows // NS
    pltpu.sync_copy(acc_sh.at[pl.ds(sid * rpt, rpt)],
                    agg_hbm.at[pl.ds(sid * rpt, rpt)])


def _gat_soft_body(src_hbm, dst_hbm, asad_hbm, ee_hbm, den_hbm,
                   src_v, dst_v, ee_v, as_v, ad_v, den_v, red_v, red2_v,
                   den_sh, den2_sh):
    sid = lax.axis_index("s")

    pltpu.sync_copy(src_hbm.at[sid], src_v)
    pltpu.sync_copy(dst_hbm.at[sid], dst_v)
    pltpu.sync_copy(asad_hbm.at[0], as_v)
    pltpu.sync_copy(asad_hbm.at[8], ad_v)

    def zden(i, _):
        den_v[pl.ds(i * 16, 16)] = jnp.zeros((16,), jnp.float32)
        return 0
    lax.fori_loop(0, NP // 16, zden, 0)

    def chunk_a(b, _):
        def sub(j, _):
            s16 = src_v[b, pl.ds(j * 16, 16)]
            d16 = dst_v[b, pl.ds(j * 16, 16)]
            sc = plsc.load_gather(as_v, [s16]) + plsc.load_gather(ad_v, [d16])
            sc = jnp.where(sc >= 0, sc, 0.2 * sc)
            ee = jnp.exp(sc)
            ee_v[b, pl.ds(j * 16, 16)] = ee
            plsc.addupdate_scatter(den_v, [d16], ee)
            return 0
        lax.fori_loop(0, RB // 16, sub, 0)
        return 0
    lax.fori_loop(0, NB, chunk_a, 0)

    pltpu.sync_copy(den_v, den_sh.at[sid])
    plsc.subcore_barrier()
    pltpu.sync_copy(den_sh.at[:, pl.ds(sid * RED, RED)], red_v)

    def red_body(j, _):
        acc = red_v[0, pl.ds(j * 16, 16)]
        for t in range(1, NS):
            acc = acc + red_v[t, pl.ds(j * 16, 16)]
        red2_v[pl.ds(j * 16, 16)] = acc
        return 0
    lax.fori_loop(0, RED // 16, red_body, 0)
    pltpu.sync_copy(red2_v, den2_sh.at[pl.ds(sid * RED, RED)])

    pltpu.sync_copy(ee_v, ee_hbm.at[sid])
    pltpu.sync_copy(red2_v, den_hbm.at[pl.ds(sid * RED, RED)])


def _gat_agg_body(src_hbm, dst_hbm, ee_hbm, den_hbm, h_hbm, agg_hbm,
                  src_v, dst_v, ee_v, den_v, rows0, rows1,
                  alpha_v, dstl_v, acc_sh, gs0, gs1,
                  *, lo, nrows):
    sid = lax.axis_index("s")

    pltpu.sync_copy(src_hbm.at[sid], src_v)
    pltpu.sync_copy(dst_hbm.at[sid], dst_v)
    pltpu.sync_copy(ee_hbm.at[sid], ee_v)
    pltpu.sync_copy(den_hbm, den_v)

    _zero_acc_slice(sid, rows0, acc_sh, nrows // NS)
    plsc.subcore_barrier()

    _phase_b(sid, src_v, dst_v, ee_v, den_v, rows0, rows1, gs0, gs1,
             alpha_v, dstl_v, acc_sh, h_hbm, agg_hbm, lo, nrows)


@functools.lru_cache(maxsize=1)
def _build_sc_kernels():
    mesh = plsc.VectorSubcoreMesh(core_axis_name="c", subcore_axis_name="s",
                                  num_cores=1)
    f32 = jnp.float32
    agg_t = jax.ShapeDtypeStruct((NP, HH), f32)
    common_v = [
        pltpu.VMEM((NB, RB), jnp.int32),
        pltpu.VMEM((NB, RB), jnp.int32),
        pltpu.VMEM((NB, RB), f32),
    ]
    NSPLIT = 5120
    soft = functools.partial(
        pl.kernel,
        mesh=mesh,
        out_type=[jax.ShapeDtypeStruct((NS, NB, RB), f32),
                  jax.ShapeDtypeStruct((NP,), f32)],
        scratch_types=common_v + [
            pltpu.VMEM((NP,), f32),
            pltpu.VMEM((NP,), f32),
            pltpu.VMEM((NP,), f32),
            pltpu.VMEM((NS, RED), f32),
            pltpu.VMEM((RED,), f32),
            pltpu.VMEM_SHARED((NS, NP), f32),
            pltpu.VMEM_SHARED((NP,), f32),
        ],
        compiler_params=pltpu.CompilerParams(needs_layout_passes=False),
    )(_gat_soft_body)

    def make_agg(lo, nrows):
        body = functools.partial(_gat_agg_body, lo=lo, nrows=nrows)
        return functools.partial(
            pl.kernel,
            mesh=mesh,
            out_type=[jax.ShapeDtypeStruct((nrows, HH), f32)],
            scratch_types=common_v + [
                pltpu.VMEM((NP,), f32),
                pltpu.VMEM((RB, HH), f32),
                pltpu.VMEM((RB, HH), f32),
                pltpu.VMEM((RB,), f32),
                pltpu.VMEM((8, RB), jnp.int32),
                pltpu.VMEM_SHARED((nrows, HH), f32),
                pltpu.SemaphoreType.DMA,
                pltpu.SemaphoreType.DMA,
            ],
            compiler_params=pltpu.CompilerParams(needs_layout_passes=False),
        )(body)

    return soft, make_agg(0, NSPLIT), make_agg(NSPLIT, NP - NSPLIT)


def _gat_edge(src2, dst2, asad, hl, hr):
    soft, agg_a, agg_b = _build_sc_kernels()
    ee, den = soft(src2, dst2, asad)
    (aggl_a,) = agg_a(src2, dst2, ee, den, hl)
    (aggr_a,) = agg_a(src2, dst2, ee, den, hr)
    (aggl_b,) = agg_b(src2, dst2, ee, den, hl)
    (aggr_b,) = agg_b(src2, dst2, ee, den, hr)
    aggl = jnp.concatenate([aggl_a, aggl_b], axis=0)
    aggr = jnp.concatenate([aggr_a, aggr_b], axis=0)
    return aggl, aggr


def kernel(X_t, t, edge_index, batch, emb_t, W_in, b_in, W1, a_src1, a_dst1,
           b1, W2, a_src2, a_dst2, b2, W_out, b_out):
    f32 = jnp.float32

    xp = jnp.pad(X_t, ((0, NP - N), (0, 5)))
    wi = jnp.pad(W_in, ((0, 5), (0, 0)))
    bat2 = jnp.pad(batch, (0, NP - N)).reshape(NP, 1).astype(jnp.int32)
    t2 = t.reshape(B, 1).astype(jnp.int32)

    loop = jnp.arange(N, dtype=jnp.int32)
    epad = jnp.full((EP - E_TOT,), PN, jnp.int32)
    src2 = jnp.concatenate([edge_index[0].astype(jnp.int32), loop, epad]
                           ).reshape(NS, NB, RB)
    dst2 = jnp.concatenate([edge_index[1].astype(jnp.int32), loop, epad]
                           ).reshape(NS, NB, RB)

    wo = jnp.pad(W_out, ((0, 0), (0, 125)))
    bo = jnp.pad(b_out, (0, 125))[None, :]

    temb = pl.pallas_call(
        _temb_body,
        out_shape=jax.ShapeDtypeStruct((B, H), f32),
    )(t2, emb_t)

    full = lambda s: pl.BlockSpec(s, lambda i: (0, 0))
    rowb = lambda s: pl.BlockSpec(s, lambda i: (i, 0))
    asadb = pl.BlockSpec((16, BLK), lambda i: (0, i))
    hout = [jax.ShapeDtypeStruct((NP, HH), f32)] * 2 + [
        jax.ShapeDtypeStruct((16, NP), f32)]
    hspecs = [rowb((BLK, HH))] * 2 + [asadb]

    hl1, hr1, asad1 = pl.pallas_call(
        _dense1_body,
        grid=(NBLK,),
        in_specs=[rowb((BLK, 8)), rowb((BLK, 1)), full((B, H)), full((8, H)),
                  full((1, H)), full((H, H)), full((1, H)), full((1, H))],
        out_specs=hspecs,
        out_shape=hout,
    )(xp, bat2, temb, wi, b_in[None, :], W1, a_src1[None, :], a_dst1[None, :])

    aggl1, aggr1 = _gat_edge(src2, dst2, asad1, hl1, hr1)

    hl2, hr2, asad2 = pl.pallas_call(
        _dense2_body,
        grid=(NBLK,),
        in_specs=[rowb((BLK, HH))] * 2 + [full((1, H)), full((H, H)),
                                          full((1, H)), full((1, H))],
        out_specs=hspecs,
        out_shape=hout,
    )(aggl1, aggr1, b1[None, :], W2, a_src2[None, :], a_dst2[None, :])

    aggl2, aggr2 = _gat_edge(src2, dst2, asad2, hl2, hr2)

    y = pl.pallas_call(
        _dense3_body,
        grid=(NBLK,),
        in_specs=[rowb((BLK, HH))] * 2 + [full((1, H)), full((H, 128)),
                                          full((1, 128))],
        out_specs=rowb((BLK, 128)),
        out_shape=jax.ShapeDtypeStruct((NP, 128), f32),
    )(aggl2, aggr2, b2[None, :], wo, bo)

    return y[:N, :3]


# speedup vs baseline: 2.4009x; 1.0028x over previous
import functools

import jax
import jax.numpy as jnp
from jax import lax
from jax.experimental import pallas as pl
from jax.experimental.pallas import tpu as pltpu
from jax.experimental.pallas import tpu_sc as plsc

N = 10000
H = 256
B = 64
T = 1000

NP = 10240
BLK = 512
NBLK = NP // BLK
PN = N

NS = 16
HH = 128
RB = 128
NB = 84
CE = NB * RB
EP = NS * CE
RED = NP // NS
E_TOT = 160000 + N


def _temb_body(t_ref, emb_ref, out_ref):
    tt = t_ref[...]
    oh = (lax.broadcasted_iota(jnp.int32, (B, T), 1) == tt).astype(jnp.float32)
    out_ref[...] = jnp.dot(oh, emb_ref[...], preferred_element_type=jnp.float32)


def _scores(h, asrc, adst):
    a_s = lax.dot_general(asrc, h, (((1,), (1,)), ((), ())),
                          preferred_element_type=jnp.float32)
    a_d = lax.dot_general(adst, h, (((1,), (1,)), ((), ())),
                          preferred_element_type=jnp.float32)
    z = jnp.zeros((7, BLK), jnp.float32)
    return jnp.concatenate([a_s, z, a_d, z], axis=0)


def _dense1_body(x_ref, bat_ref, temb_ref, wi_ref, bi_ref, w_ref, asrc_ref,
                 adst_ref, hl_ref, hr_ref, asad_ref):
    bidx = bat_ref[...]
    oh = (lax.broadcasted_iota(jnp.int32, (BLK, B), 1) == bidx).astype(jnp.float32)
    x = (jnp.dot(x_ref[...], wi_ref[...], preferred_element_type=jnp.float32)
         + bi_ref[...]
         + jnp.dot(oh, temb_ref[...], preferred_element_type=jnp.float32))
    h = jnp.dot(x, w_ref[...], preferred_element_type=jnp.float32)
    hl_ref[...] = h[:, :HH]
    hr_ref[...] = h[:, HH:]
    asad_ref[...] = _scores(h, asrc_ref[...], adst_ref[...])


def _dense2_body(al_ref, ar_ref, b1_ref, w_ref, asrc_ref, adst_ref,
                 hl_ref, hr_ref, asad_ref):
    x = jnp.concatenate([al_ref[...], ar_ref[...]], axis=1) + b1_ref[...]
    x = jnp.maximum(x, 0.0)
    h = jnp.dot(x, w_ref[...], preferred_element_type=jnp.float32)
    hl_ref[...] = h[:, :HH]
    hr_ref[...] = h[:, HH:]
    asad_ref[...] = _scores(h, asrc_ref[...], adst_ref[...])


def _dense3_body(al_ref, ar_ref, b2_ref, wo_ref, bo_ref, y_ref):
    x = jnp.concatenate([al_ref[...], ar_ref[...]], axis=1) + b2_ref[...]
    y_ref[...] = jnp.dot(x, wo_ref[...], preferred_element_type=jnp.float32) + bo_ref[...]


ZB = 64


def _zero_acc_slice(sid, zbuf, acc_sh, rpt):
    def row(i, _):
        for c in range(HH // 16):
            zbuf[i, pl.ds(c * 16, 16)] = jnp.zeros((16,), jnp.float32)
        return 0
    lax.fori_loop(0, ZB, row, 0)
    for k in range(rpt // ZB):
        pltpu.sync_copy(zbuf.at[pl.ds(0, ZB)],
                        acc_sh.at[pl.ds(sid * rpt + k * ZB, ZB)])


def _phase_b(sid, src_v, dst_v, ee_v, den_v, rows0, rows1, gs0, gs1,
             alpha_v, dstl_v, acc_sh, h_hbm, agg_hbm, lo, nrows):
    pltpu.async_copy(h_hbm.at[src_v.at[0]], rows0, gs0)
    pltpu.async_copy(h_hbm.at[src_v.at[1]], rows1, gs1)

    def pair(p, _):
        for k, rows_v, gsem in ((0, rows0, gs0), (1, rows1, gs1)):
            b = 2 * p + k

            def asub(j, _):
                d16 = dst_v[b, pl.ds(j * 16, 16)]
                den = plsc.load_gather(den_v, [d16])
                ee = ee_v[b, pl.ds(j * 16, 16)]
                dloc = d16 - lo
                inr = (dloc >= 0) & (dloc < nrows)
                alpha_v[pl.ds(j * 16, 16)] = jnp.where(
                    inr, ee / (den + 1e-16), 0.0)
                dstl_v[0, pl.ds(j * 16, 16)] = jnp.where(inr, dloc, 0)
                return 0
            lax.fori_loop(0, RB // 16, asub, 0)

            pltpu.make_async_copy(h_hbm.at[src_v.at[b]], rows_v, gsem).wait()

            def rsub(g, _):
                a16 = alpha_v[pl.ds(g * 16, 16)]
                for l in range(16):
                    av = jnp.full((16,), a16[l], jnp.float32)
                    r = g * 16 + l
                    for c in range(HH // 16):
                        rows_v[r, pl.ds(c * 16, 16)] = (
                            rows_v[r, pl.ds(c * 16, 16)] * av)
                return 0
            lax.fori_loop(0, RB // 16, rsub, 0)

            pltpu.sync_copy(rows_v, acc_sh.at[dstl_v.at[0]], add=True)

            @pl.when(b + 2 < NB)
            def _():
                pltpu.async_copy(h_hbm.at[src_v.at[b + 2]], rows_v, gsem)
        return 0
    lax.fori_loop(0, NB // 2, pair, 0)

    plsc.subcore_barrier()
    rpt = nrows // NS
    pltpu.sync_copy(acc_sh.at[pl.ds(sid * rpt, rpt)],
                    agg_hbm.at[pl.ds(sid * rpt, rpt)])


def _gat_soft_body(src_hbm, dst_hbm, asad_hbm, ee_hbm, den_hbm,
                   src_v, dst_v, ee_v, as_v, ad_v, den_v, red_v, red2_v,
                   den_sh, den2_sh):
    sid = lax.axis_index("s")

    pltpu.sync_copy(src_hbm.at[sid], src_v)
    pltpu.sync_copy(dst_hbm.at[sid], dst_v)
    pltpu.sync_copy(asad_hbm.at[0], as_v)
    pltpu.sync_copy(asad_hbm.at[8], ad_v)

    def zden(i, _):
        den_v[pl.ds(i * 16, 16)] = jnp.zeros((16,), jnp.float32)
        return 0
    lax.fori_loop(0, NP // 16, zden, 0)

    def chunk_a(b, _):
        def sub(j, _):
            s16 = src_v[b, pl.ds(j * 16, 16)]
            d16 = dst_v[b, pl.ds(j * 16, 16)]
            sc = plsc.load_gather(as_v, [s16]) + plsc.load_gather(ad_v, [d16])
            sc = jnp.where(sc >= 0, sc, 0.2 * sc)
            ee = jnp.exp(sc)
            ee_v[b, pl.ds(j * 16, 16)] = ee
            plsc.addupdate_scatter(den_v, [d16], ee)
            return 0
        lax.fori_loop(0, RB // 16, sub, 0)
        return 0
    lax.fori_loop(0, NB, chunk_a, 0)

    pltpu.sync_copy(den_v, den_sh.at[sid])
    plsc.subcore_barrier()
    pltpu.sync_copy(den_sh.at[:, pl.ds(sid * RED, RED)], red_v)

    def red_body(j, _):
        acc = red_v[0, pl.ds(j * 16, 16)]
        for t in range(1, NS):
            acc = acc + red_v[t, pl.ds(j * 16, 16)]
        red2_v[pl.ds(j * 16, 16)] = acc
        return 0
    lax.fori_loop(0, RED // 16, red_body, 0)
    pltpu.sync_copy(red2_v, den2_sh.at[pl.ds(sid * RED, RED)])

    pltpu.sync_copy(ee_v, ee_hbm.at[sid])
    pltpu.sync_copy(red2_v, den_hbm.at[pl.ds(sid * RED, RED)])


def _gat_agg_body(src_hbm, dst_hbm, ee_hbm, den_hbm, h_hbm, agg_hbm,
                  src_v, dst_v, ee_v, den_v, rows0, rows1,
                  alpha_v, dstl_v, acc_sh, gs0, gs1,
                  *, lo, nrows):
    sid = lax.axis_index("s")

    pltpu.sync_copy(src_hbm.at[sid], src_v)
    pltpu.sync_copy(dst_hbm.at[sid], dst_v)
    pltpu.sync_copy(ee_hbm.at[sid], ee_v)
    pltpu.sync_copy(den_hbm, den_v)

    _zero_acc_slice(sid, rows0, acc_sh, nrows // NS)
    plsc.subcore_barrier()

    _phase_b(sid, src_v, dst_v, ee_v, den_v, rows0, rows1, gs0, gs1,
             alpha_v, dstl_v, acc_sh, h_hbm, agg_hbm, lo, nrows)


@functools.lru_cache(maxsize=1)
def _build_sc_kernels():
    mesh = plsc.VectorSubcoreMesh(core_axis_name="c", subcore_axis_name="s",
                                  num_cores=1)
    f32 = jnp.float32
    agg_t = jax.ShapeDtypeStruct((NP, HH), f32)
    common_v = [
        pltpu.VMEM((NB, RB), jnp.int32),
        pltpu.VMEM((NB, RB), jnp.int32),
        pltpu.VMEM((NB, RB), f32),
    ]
    NSPLIT = 5120
    soft = functools.partial(
        pl.kernel,
        mesh=mesh,
        out_type=[jax.ShapeDtypeStruct((NS, NB, RB), f32),
                  jax.ShapeDtypeStruct((NP,), f32)],
        scratch_types=common_v + [
            pltpu.VMEM((NP,), f32),
            pltpu.VMEM((NP,), f32),
            pltpu.VMEM((NP,), f32),
            pltpu.VMEM((NS, RED), f32),
            pltpu.VMEM((RED,), f32),
            pltpu.VMEM_SHARED((NS, NP), f32),
            pltpu.VMEM_SHARED((NP,), f32),
        ],
        compiler_params=pltpu.CompilerParams(needs_layout_passes=False),
    )(_gat_soft_body)

    def make_agg(lo, nrows):
        body = functools.partial(_gat_agg_body, lo=lo, nrows=nrows)
        return functools.partial(
            pl.kernel,
            mesh=mesh,
            out_type=[jax.ShapeDtypeStruct((nrows, HH), f32)],
            scratch_types=common_v + [
                pltpu.VMEM((NP,), f32),
                pltpu.VMEM((RB, HH), f32),
                pltpu.VMEM((RB, HH), f32),
                pltpu.VMEM((RB,), f32),
                pltpu.VMEM((8, RB), jnp.int32),
                pltpu.VMEM_SHARED((nrows, HH), f32),
                pltpu.SemaphoreType.DMA,
                pltpu.SemaphoreType.DMA,
            ],
            compiler_params=pltpu.CompilerParams(needs_layout_passes=False),
        )(body)

    return soft, make_agg(0, NSPLIT), make_agg(NSPLIT, NP - NSPLIT)


def _gat_edge(src2, dst2, asad, hl, hr):
    soft, agg_a, agg_b = _build_sc_kernels()
    ee, den = soft(src2, dst2, asad)
    (aggl_a,) = agg_a(src2, dst2, ee, den, hl)
    (aggr_a,) = agg_a(src2, dst2, ee, den, hr)
    (aggl_b,) = agg_b(src2, dst2, ee, den, hl)
    (aggr_b,) = agg_b(src2, dst2, ee, den, hr)
    aggl = jnp.concatenate([aggl_a, aggl_b], axis=0)
    aggr = jnp.concatenate([aggr_a, aggr_b], axis=0)
    return aggl, aggr


def kernel(X_t, t, edge_index, batch, emb_t, W_in, b_in, W1, a_src1, a_dst1,
           b1, W2, a_src2, a_dst2, b2, W_out, b_out):
    f32 = jnp.float32

    xp = jnp.pad(X_t, ((0, NP - N), (0, 5)))
    wi = jnp.pad(W_in, ((0, 5), (0, 0)))
    bat2 = jnp.pad(batch, (0, NP - N)).reshape(NP, 1).astype(jnp.int32)
    t2 = t.reshape(B, 1).astype(jnp.int32)

    loop = jnp.arange(N, dtype=jnp.int32)
    epad = jnp.full((EP - E_TOT,), PN, jnp.int32)
    src2 = jnp.concatenate([edge_index[0].astype(jnp.int32), loop, epad]
                           ).reshape(NS, NB, RB)
    dst2 = jnp.concatenate([edge_index[1].astype(jnp.int32), loop, epad]
                           ).reshape(NS, NB, RB)

    wo = jnp.pad(W_out, ((0, 0), (0, 125)))
    bo = jnp.pad(b_out, (0, 125))[None, :]

    temb = pl.pallas_call(
        _temb_body,
        out_shape=jax.ShapeDtypeStruct((B, H), f32),
    )(t2, emb_t)

    full = lambda s: pl.BlockSpec(s, lambda i: (0, 0))
    rowb = lambda s: pl.BlockSpec(s, lambda i: (i, 0))
    asadb = pl.BlockSpec((16, BLK), lambda i: (0, i))
    hout = [jax.ShapeDtypeStruct((NP, HH), f32)] * 2 + [
        jax.ShapeDtypeStruct((16, NP), f32)]
    hspecs = [rowb((BLK, HH))] * 2 + [asadb]

    hl1, hr1, asad1 = pl.pallas_call(
        _dense1_body,
        grid=(NBLK,),
        in_specs=[rowb((BLK, 8)), rowb((BLK, 1)), full((B, H)), full((8, H)),
                  full((1, H)), full((H, H)), full((1, H)), full((1, H))],
        out_specs=hspecs,
        out_shape=hout,
    )(xp, bat2, temb, wi, b_in[None, :], W1, a_src1[None, :], a_dst1[None, :])

    aggl1, aggr1 = _gat_edge(src2, dst2, asad1, hl1, hr1)

    hl2, hr2, asad2 = pl.pallas_call(
        _dense2_body,
        grid=(NBLK,),
        in_specs=[rowb((BLK, HH))] * 2 + [full((1, H)), full((H, H)),
                                          full((1, H)), full((1, H))],
        out_specs=hspecs,
        out_shape=hout,
    )(aggl1, aggr1, b1[None, :], W2, a_src2[None, :], a_dst2[None, :])

    aggl2, aggr2 = _gat_edge(src2, dst2, asad2, hl2, hr2)

    y = pl.pallas_call(
        _dense3_body,
        grid=(NBLK,),
        in_specs=[rowb((BLK, HH))] * 2 + [full((1, H)), full((H, 128)),
                                          full((1, 128))],
        out_specs=rowb((BLK, 128)),
        out_shape=jax.ShapeDtypeStruct((NP, 128), f32),
    )(aggl2, aggr2, b2[None, :], wo, bo)

    return y[:N, :3]
